# baseline (speedup 1.0000x reference)
import jax
import jax.numpy as jnp
from jax.experimental import pallas as pl

TOPK = 64
ROWS_PER_BLOCK = 64
SEARCH_BITS = 31


def _count_ge(b, mid):
    return jnp.sum((b >= mid).astype(jnp.int32), axis=1, keepdims=True)


def _abs_topk_block(x_ref, o_ref):
    x = x_ref[...]
    b = jax.lax.bitcast_convert_type(x, jnp.int32) & jnp.int32(0x7FFFFFFF)
    r, n = x.shape

    def body(_, lohi):
        lo, hi = lohi
        mid = lo + ((hi - lo) >> 1)
        ge = _count_ge(b, mid) >= TOPK
        return jnp.where(ge, mid, lo), jnp.where(ge, hi, mid)

    lo0 = jnp.zeros((r, 1), jnp.int32)
    hi0 = jnp.full((r, 1), jnp.int32(0x7FFFFFFF))
    t, _ = jax.lax.fori_loop(0, SEARCH_BITS, body, (lo0, hi0), unroll=True)

    gt = b > t
    eq = b == t
    n_gt = jnp.sum(gt.astype(jnp.int32), axis=1, keepdims=True)
    n_eq = jnp.sum(eq.astype(jnp.int32), axis=1, keepdims=True)
    n_eq_keep = TOPK - n_gt

    idx = jax.lax.broadcasted_iota(jnp.int32, b.shape, 1)

    def tie_cutoff():
        def body_idx(_, lohi):
            lo, hi = lohi
            mid = lo + ((hi - lo) >> 1)
            cnt = jnp.sum((eq & (idx < mid)).astype(jnp.int32), axis=1,
                          keepdims=True)
            ge = cnt >= n_eq_keep
            return jnp.where(ge, lo, mid), jnp.where(ge, mid, hi)

        lo0 = jnp.zeros_like(n_gt)
        hi0 = jnp.full_like(n_gt, b.shape[1])
        _, cut = jax.lax.fori_loop(0, 15, body_idx, (lo0, hi0), unroll=True)
        return cut

    cut = jax.lax.cond(jnp.any(n_eq > n_eq_keep), tie_cutoff,
                       lambda: jnp.full_like(n_gt, b.shape[1]))
    keep = gt | (eq & (idx < cut))
    o_ref[...] = jnp.where(keep, x, jnp.float32(0.0))


@jax.jit
def kernel(x):
    m, n = x.shape
    return pl.pallas_call(
        _abs_topk_block,
        grid=(m // ROWS_PER_BLOCK,),
        in_specs=[pl.BlockSpec((ROWS_PER_BLOCK, n), lambda i: (i, 0))],
        out_specs=pl.BlockSpec((ROWS_PER_BLOCK, n), lambda i: (i, 0)),
        out_shape=jax.ShapeDtypeStruct((m, n), x.dtype),
    )(x)


# speedup vs baseline: 1.1634x; 1.1634x over previous
import jax
import jax.numpy as jnp
from jax.experimental import pallas as pl

TOPK = 64
ROWS_PER_BLOCK = 16
SEARCH_BITS = 31


def _count_ge(b, mid):
    return jnp.sum((b >= mid).astype(jnp.int32), axis=1, keepdims=True)


def _abs_topk_block(x_ref, o_ref):
    x = x_ref[...]
    b = jax.lax.bitcast_convert_type(x, jnp.int32) & jnp.int32(0x7FFFFFFF)
    r, n = x.shape

    def body(_, lohi):
        lo, hi = lohi
        mid = lo + ((hi - lo) >> 1)
        ge = _count_ge(b, mid) >= TOPK
        return jnp.where(ge, mid, lo), jnp.where(ge, hi, mid)

    lo0 = jnp.zeros((r, 1), jnp.int32)
    hi0 = jnp.full((r, 1), jnp.int32(0x7FFFFFFF))
    t, _ = jax.lax.fori_loop(0, SEARCH_BITS, body, (lo0, hi0), unroll=True)

    gt = b > t
    eq = b == t
    n_gt = jnp.sum(gt.astype(jnp.int32), axis=1, keepdims=True)
    n_eq = jnp.sum(eq.astype(jnp.int32), axis=1, keepdims=True)
    n_eq_keep = TOPK - n_gt

    idx = jax.lax.broadcasted_iota(jnp.int32, b.shape, 1)

    def tie_cutoff():
        def body_idx(_, lohi):
            lo, hi = lohi
            mid = lo + ((hi - lo) >> 1)
            cnt = jnp.sum((eq & (idx < mid)).astype(jnp.int32), axis=1,
                          keepdims=True)
            ge = cnt >= n_eq_keep
            return jnp.where(ge, lo, mid), jnp.where(ge, mid, hi)

        lo0 = jnp.zeros_like(n_gt)
        hi0 = jnp.full_like(n_gt, b.shape[1])
        _, cut = jax.lax.fori_loop(0, 15, body_idx, (lo0, hi0), unroll=True)
        return cut

    cut = jax.lax.cond(jnp.any(n_eq > n_eq_keep), tie_cutoff,
                       lambda: jnp.full_like(n_gt, b.shape[1]))
    keep = gt | (eq & (idx < cut))
    o_ref[...] = jnp.where(keep, x, jnp.float32(0.0))


@jax.jit
def kernel(x):
    m, n = x.shape
    return pl.pallas_call(
        _abs_topk_block,
        grid=(m // ROWS_PER_BLOCK,),
        in_specs=[pl.BlockSpec((ROWS_PER_BLOCK, n), lambda i: (i, 0))],
        out_specs=pl.BlockSpec((ROWS_PER_BLOCK, n), lambda i: (i, 0)),
        out_shape=jax.ShapeDtypeStruct((m, n), x.dtype),
    )(x)


# speedup vs baseline: 1.3224x; 1.1367x over previous
import jax
import jax.numpy as jnp
from jax.experimental import pallas as pl

TOPK = 64
ROWS_PER_BLOCK = 32
SEARCH_BITS = 31


def _count_ge(b, mid):
    return jnp.sum((b >= mid).astype(jnp.int32), axis=1, keepdims=True)


def _abs_topk_block(x_ref, o_ref):
    x = x_ref[...]
    b = jax.lax.bitcast_convert_type(x, jnp.int32) & jnp.int32(0x7FFFFFFF)
    r, n = x.shape

    m = jnp.max(b.reshape(r, n // 128, 128), axis=1)

    def body_m(_, lohi):
        lo, hi = lohi
        mid = lo + ((hi - lo) >> 1)
        ge = _count_ge(m, mid) >= TOPK
        return jnp.where(ge, mid, lo), jnp.where(ge, hi, mid)

    t_lo, _ = jax.lax.fori_loop(
        0, SEARCH_BITS, body_m,
        (jnp.zeros((r, 1), jnp.int32), jnp.full((r, 1), jnp.int32(0x7FFFFFFF))),
        unroll=True)
    gmax = jnp.max(m, axis=1, keepdims=True)
    hi0 = jnp.where(gmax == jnp.int32(0x7FFFFFFF), gmax, gmax + 1)
    hi0 = jnp.maximum(hi0, t_lo + 1)

    def body(_, lohi):
        lo, hi = lohi
        mid = lo + ((hi - lo) >> 1)
        ge = _count_ge(b, mid) >= TOPK
        return jnp.where(ge, mid, lo), jnp.where(ge, hi, mid)

    lo1, hi1 = jax.lax.fori_loop(0, 24, body, (t_lo, hi0), unroll=True)
    t, _ = jax.lax.cond(
        jnp.any(hi1 - lo1 > 1),
        lambda: jax.lax.fori_loop(0, SEARCH_BITS - 24, body, (lo1, hi1),
                                  unroll=True),
        lambda: (lo1, hi1))

    gt = b > t
    eq = b == t
    n_gt = jnp.sum(gt.astype(jnp.int32), axis=1, keepdims=True)
    n_eq = jnp.sum(eq.astype(jnp.int32), axis=1, keepdims=True)
    n_eq_keep = TOPK - n_gt

    idx = jax.lax.broadcasted_iota(jnp.int32, b.shape, 1)

    def tie_cutoff():
        def body_idx(_, lohi):
            lo, hi = lohi
            mid = lo + ((hi - lo) >> 1)
            cnt = jnp.sum((eq & (idx < mid)).astype(jnp.int32), axis=1,
                          keepdims=True)
            ge = cnt >= n_eq_keep
            return jnp.where(ge, lo, mid), jnp.where(ge, mid, hi)

        lo0 = jnp.zeros_like(n_gt)
        hi0 = jnp.full_like(n_gt, b.shape[1])
        _, cut = jax.lax.fori_loop(0, 15, body_idx, (lo0, hi0), unroll=True)
        return cut

    cut = jax.lax.cond(jnp.any(n_eq > n_eq_keep), tie_cutoff,
                       lambda: jnp.full_like(n_gt, b.shape[1]))
    keep = gt | (eq & (idx < cut))
    o_ref[...] = jnp.where(keep, x, jnp.float32(0.0))


@jax.jit
def kernel(x):
    m, n = x.shape
    return pl.pallas_call(
        _abs_topk_block,
        grid=(m // ROWS_PER_BLOCK,),
        in_specs=[pl.BlockSpec((ROWS_PER_BLOCK, n), lambda i: (i, 0))],
        out_specs=pl.BlockSpec((ROWS_PER_BLOCK, n), lambda i: (i, 0)),
        out_shape=jax.ShapeDtypeStruct((m, n), x.dtype),
    )(x)


# speedup vs baseline: 1.4943x; 1.1300x over previous
import jax
import jax.numpy as jnp
from jax.experimental import pallas as pl

TOPK = 64
ROWS_PER_BLOCK = 32
SEARCH_BITS = 31


def _count_ge(b, mid):
    return jnp.sum((b >= mid).astype(jnp.int32), axis=1, keepdims=True)


def _abs_topk_block(x_ref, o_ref):
    x = x_ref[...]
    b = jax.lax.bitcast_convert_type(x, jnp.int32) & jnp.int32(0x7FFFFFFF)
    r, n = x.shape

    m = jnp.max(b.reshape(r, n // 128, 128), axis=1)
    t_lo = jnp.min(m, axis=1, keepdims=True)
    gmax = jnp.max(m, axis=1, keepdims=True)
    hi0 = jnp.where(gmax == jnp.int32(0x7FFFFFFF), gmax, gmax + 1)
    hi0 = jnp.maximum(hi0, t_lo + 1)

    def body(_, lohi):
        lo, hi = lohi
        mid = lo + ((hi - lo) >> 1)
        ge = _count_ge(b, mid) >= TOPK
        return jnp.where(ge, mid, lo), jnp.where(ge, hi, mid)

    lo1, hi1 = jax.lax.fori_loop(0, 25, body, (t_lo, hi0), unroll=True)
    t, _ = jax.lax.cond(
        jnp.any(hi1 - lo1 > 1),
        lambda: jax.lax.fori_loop(0, SEARCH_BITS - 25, body, (lo1, hi1),
                                  unroll=True),
        lambda: (lo1, hi1))

    gt = b > t
    eq = b == t
    n_gt = jnp.sum(gt.astype(jnp.int32), axis=1, keepdims=True)
    n_eq = jnp.sum(eq.astype(jnp.int32), axis=1, keepdims=True)
    n_eq_keep = TOPK - n_gt

    idx = jax.lax.broadcasted_iota(jnp.int32, b.shape, 1)

    def tie_cutoff():
        def body_idx(_, lohi):
            lo, hi = lohi
            mid = lo + ((hi - lo) >> 1)
            cnt = jnp.sum((eq & (idx < mid)).astype(jnp.int32), axis=1,
                          keepdims=True)
            ge = cnt >= n_eq_keep
            return jnp.where(ge, lo, mid), jnp.where(ge, mid, hi)

        lo0 = jnp.zeros_like(n_gt)
        hi0 = jnp.full_like(n_gt, b.shape[1])
        _, cut = jax.lax.fori_loop(0, 15, body_idx, (lo0, hi0), unroll=True)
        return cut

    cut = jax.lax.cond(jnp.any(n_eq > n_eq_keep), tie_cutoff,
                       lambda: jnp.full_like(n_gt, b.shape[1]))
    keep = gt | (eq & (idx < cut))
    o_ref[...] = jnp.where(keep, x, jnp.float32(0.0))


@jax.jit
def kernel(x):
    m, n = x.shape
    return pl.pallas_call(
        _abs_topk_block,
        grid=(m // ROWS_PER_BLOCK,),
        in_specs=[pl.BlockSpec((ROWS_PER_BLOCK, n), lambda i: (i, 0))],
        out_specs=pl.BlockSpec((ROWS_PER_BLOCK, n), lambda i: (i, 0)),
        out_shape=jax.ShapeDtypeStruct((m, n), x.dtype),
    )(x)


# speedup vs baseline: 1.5396x; 1.0303x over previous
import jax
import jax.numpy as jnp
from jax.experimental import pallas as pl

TOPK = 64
ROWS_PER_BLOCK = 32
SEARCH_BITS = 31


def _count_ge(b, mid):
    return jnp.sum((b >= mid).astype(jnp.int32), axis=1, keepdims=True)


def _abs_topk_block(x_ref, o_ref):
    x = x_ref[...]
    b = jax.lax.bitcast_convert_type(x, jnp.int32) & jnp.int32(0x7FFFFFFF)
    r, n = x.shape

    m = jnp.max(b.reshape(r, n // 128, 128), axis=1)
    t_lo = jnp.min(m, axis=1, keepdims=True)
    gmax = jnp.max(m, axis=1, keepdims=True)
    hi0 = jnp.where(gmax == jnp.int32(0x7FFFFFFF), gmax, gmax + 1)
    hi0 = jnp.maximum(hi0, t_lo + 1)

    def body(_, lohi):
        lo, hi = lohi
        mid = lo + ((hi - lo) >> 1)
        ge = _count_ge(b, mid) >= TOPK
        return jnp.where(ge, mid, lo), jnp.where(ge, hi, mid)

    lo1, hi1 = jax.lax.fori_loop(0, 24, body, (t_lo, hi0), unroll=True)
    t, _ = jax.lax.cond(
        jnp.any(hi1 - lo1 > 1),
        lambda: jax.lax.fori_loop(0, SEARCH_BITS - 24, body, (lo1, hi1),
                                  unroll=True),
        lambda: (lo1, hi1))

    gt = b > t
    eq = b == t
    n_gt = jnp.sum(gt.astype(jnp.int32), axis=1, keepdims=True)
    n_eq = jnp.sum(eq.astype(jnp.int32), axis=1, keepdims=True)
    n_eq_keep = TOPK - n_gt

    idx = jax.lax.broadcasted_iota(jnp.int32, b.shape, 1)

    def tie_cutoff():
        def body_idx(_, lohi):
            lo, hi = lohi
            mid = lo + ((hi - lo) >> 1)
            cnt = jnp.sum((eq & (idx < mid)).astype(jnp.int32), axis=1,
                          keepdims=True)
            ge = cnt >= n_eq_keep
            return jnp.where(ge, lo, mid), jnp.where(ge, mid, hi)

        lo0 = jnp.zeros_like(n_gt)
        hi0 = jnp.full_like(n_gt, b.shape[1])
        _, cut = jax.lax.fori_loop(0, 15, body_idx, (lo0, hi0), unroll=True)
        return cut

    cut = jax.lax.cond(jnp.any(n_eq > n_eq_keep), tie_cutoff,
                       lambda: jnp.full_like(n_gt, b.shape[1]))
    keep = gt | (eq & (idx < cut))
    o_ref[...] = jnp.where(keep, x, jnp.float32(0.0))


@jax.jit
def kernel(x):
    m, n = x.shape
    return pl.pallas_call(
        _abs_topk_block,
        grid=(m // ROWS_PER_BLOCK,),
        in_specs=[pl.BlockSpec((ROWS_PER_BLOCK, n), lambda i: (i, 0))],
        out_specs=pl.BlockSpec((ROWS_PER_BLOCK, n), lambda i: (i, 0)),
        out_shape=jax.ShapeDtypeStruct((m, n), x.dtype),
    )(x)
